# baseline (speedup 1.0000x reference)
import functools

import jax
import jax.numpy as jnp
from jax import lax
from jax.experimental import pallas as pl
from jax.experimental.pallas import tpu as pltpu
from jax.experimental.pallas import tpu_sc as plsc

NC, NS = 2, 16
NW = NC * NS


def _proj_body(x_ref, w_ref, b_ref, o_ref):
    acc = jnp.dot(x_ref[...].astype(jnp.bfloat16),
                  w_ref[...].astype(jnp.bfloat16),
                  preferred_element_type=jnp.float32) + b_ref[...]
    bf = jax.nn.relu(acc).astype(jnp.bfloat16)
    n = bf.shape[1]
    u = lax.bitcast_convert_type(bf, jnp.uint16).astype(jnp.uint32)
    word = u[:, :n // 2] | (u[:, n // 2:] << 16)
    o_ref[...] = lax.bitcast_convert_type(word, jnp.int32)


def _proj_matmul(x, w, b, bm):
    m, k = x.shape
    n = w.shape[1]
    return pl.pallas_call(
        _proj_body,
        grid=(pl.cdiv(m, bm),),
        in_specs=[
            pl.BlockSpec((bm, k), lambda i: (i, 0)),
            pl.BlockSpec((k, n), lambda i: (0, 0)),
            pl.BlockSpec((n,), lambda i: (0,)),
        ],
        out_specs=pl.BlockSpec((bm, n // 2), lambda i: (i, 0)),
        out_shape=jax.ShapeDtypeStruct((m, n // 2), jnp.int32),
    )(x, w, b)


def _fc2_body(x_ref, w_ref, b_ref, o_ref):
    xu = lax.bitcast_convert_type(x_ref[...], jnp.uint32)
    lo = (xu & jnp.uint32(0xFFFF)).astype(jnp.uint16)
    hi = (xu >> 16).astype(jnp.uint16)
    xb = jnp.concatenate(
        [lax.bitcast_convert_type(lo, jnp.bfloat16),
         lax.bitcast_convert_type(hi, jnp.bfloat16)], axis=1)
    o_ref[...] = jnp.dot(xb, w_ref[...].astype(jnp.bfloat16),
                         preferred_element_type=jnp.float32) + b_ref[...]


def _fc2_matmul(x, w, b, bm):
    m, kw = x.shape
    k, n = w.shape
    return pl.pallas_call(
        _fc2_body,
        grid=(m // bm,),
        in_specs=[
            pl.BlockSpec((bm, kw), lambda i: (i, 0)),
            pl.BlockSpec((k, n), lambda i: (0, 0)),
            pl.BlockSpec((n,), lambda i: (0,)),
        ],
        out_specs=pl.BlockSpec((bm, n), lambda i: (i, 0)),
        out_shape=jax.ShapeDtypeStruct((m, n), jnp.float32),
    )(x, w, b)


def _gather_max(proj, tok):
    V, Dw = proj.shape
    W = 16
    SENT = tok.shape[0] // W
    sent_per_w = SENT // NW
    NBUF = 4
    BODY = 8
    n_bodies = sent_per_w // BODY
    mesh = plsc.VectorSubcoreMesh(core_axis_name="c", subcore_axis_name="s")

    @functools.partial(
        pl.kernel,
        out_type=jax.ShapeDtypeStruct((SENT, Dw), jnp.int32),
        mesh=mesh,
        scratch_types=[
            pltpu.VMEM((sent_per_w * W,), jnp.int32),
            [pltpu.VMEM((W, Dw), jnp.int32) for _ in range(NBUF)],
            pltpu.VMEM((BODY, Dw), jnp.int32),
            [pltpu.SemaphoreType.DMA for _ in range(NBUF)],
        ],
    )
    def k(proj_hbm, tok_hbm, out_hbm, idx_v, bufs, out_v, sems):
        wid = lax.axis_index("s") * NC + lax.axis_index("c")
        base = wid * sent_per_w
        pltpu.sync_copy(tok_hbm.at[pl.ds(base * W, sent_per_w * W)], idx_v)

        def fire(local_s, b):
            idx = idx_v[pl.ds(local_s * W, W)]
            pltpu.async_copy(proj_hbm.at[idx], bufs[b], sems[b])

        def drain(b):
            pltpu.make_async_copy(
                proj_hbm.at[pl.ds(0, W)], bufs[b], sems[b]).wait()

        def reduce_sent(b, o_row):
            buf = bufs[b]

            @plsc.parallel_loop(0, Dw // 32, 1, unroll=4)
            def col_body(c):
                for half in range(2):
                    o = (2 * c + half) * 16
                    w0 = buf[0, pl.ds(o, 16)].astype(jnp.uint32)
                    m_hi = w0
                    m_lo = w0 & jnp.uint32(0xFFFF)
                    for t in range(1, W):
                        wt = buf[t, pl.ds(o, 16)].astype(jnp.uint32)
                        m_hi = jnp.maximum(m_hi, wt)
                        m_lo = jnp.maximum(m_lo, wt & jnp.uint32(0xFFFF))
                    out_v[o_row, pl.ds(o, 16)] = (
                        (m_hi & jnp.uint32(0xFFFF0000)) | m_lo
                    ).astype(jnp.int32)

        for p in range(NBUF - 1):
            fire(p, p)

        def body(q, _):
            s0 = q * BODY
            for u in range(BODY):
                b = u % NBUF

                @pl.when(s0 + u + NBUF - 1 < sent_per_w)
                def _():
                    fire(s0 + u + NBUF - 1, (u + NBUF - 1) % NBUF)

                drain(b)
                reduce_sent(b, u)

            pltpu.sync_copy(out_v, out_hbm.at[pl.ds(base + s0, BODY)])
            return 0

        lax.fori_loop(0, n_bodies, body, 0)

    return k(proj, tok)


def kernel(tokens, word_embd, fc1_w, fc1_b, fc2_w, fc2_b):
    proj = _proj_matmul(word_embd, fc1_w, fc1_b, 256)
    tok_flat = tokens.astype(jnp.int32).reshape(-1)
    pooled = _gather_max(proj, tok_flat)
    return _fc2_matmul(pooled, fc2_w, fc2_b, 512)


# speedup vs baseline: 1.0471x; 1.0471x over previous
import functools

import jax
import jax.numpy as jnp
from jax import lax
from jax.experimental import pallas as pl
from jax.experimental.pallas import tpu as pltpu
from jax.experimental.pallas import tpu_sc as plsc

NC, NS = 2, 16
NW = NC * NS


def _proj_body(x_ref, w_ref, b_ref, o_ref):
    acc = jnp.dot(x_ref[...].astype(jnp.bfloat16),
                  w_ref[...].astype(jnp.bfloat16),
                  preferred_element_type=jnp.float32) + b_ref[...]
    bf = jax.nn.relu(acc).astype(jnp.bfloat16)
    n = bf.shape[1]
    u = lax.bitcast_convert_type(bf, jnp.uint16).astype(jnp.uint32)
    word = u[:, :n // 2] | (u[:, n // 2:] << 16)
    o_ref[...] = lax.bitcast_convert_type(word, jnp.int32)


def _proj_matmul(x, w, b, bm):
    m, k = x.shape
    n = w.shape[1]
    return pl.pallas_call(
        _proj_body,
        grid=(pl.cdiv(m, bm),),
        in_specs=[
            pl.BlockSpec((bm, k), lambda i: (i, 0)),
            pl.BlockSpec((k, n), lambda i: (0, 0)),
            pl.BlockSpec((n,), lambda i: (0,)),
        ],
        out_specs=pl.BlockSpec((bm, n // 2), lambda i: (i, 0)),
        out_shape=jax.ShapeDtypeStruct((m, n // 2), jnp.int32),
    )(x, w, b)


def _fc2_body(x_ref, w_ref, b_ref, o_ref):
    xu = lax.bitcast_convert_type(x_ref[...], jnp.uint32)
    lo = (xu & jnp.uint32(0xFFFF)).astype(jnp.uint16)
    hi = (xu >> 16).astype(jnp.uint16)
    xb = jnp.concatenate(
        [lax.bitcast_convert_type(lo, jnp.bfloat16),
         lax.bitcast_convert_type(hi, jnp.bfloat16)], axis=1)
    o_ref[...] = jnp.dot(xb, w_ref[...].astype(jnp.bfloat16),
                         preferred_element_type=jnp.float32) + b_ref[...]


def _fc2_matmul(x, w, b, bm):
    m, kw = x.shape
    k, n = w.shape
    return pl.pallas_call(
        _fc2_body,
        grid=(m // bm,),
        in_specs=[
            pl.BlockSpec((bm, kw), lambda i: (i, 0)),
            pl.BlockSpec((k, n), lambda i: (0, 0)),
            pl.BlockSpec((n,), lambda i: (0,)),
        ],
        out_specs=pl.BlockSpec((bm, n), lambda i: (i, 0)),
        out_shape=jax.ShapeDtypeStruct((m, n), jnp.float32),
    )(x, w, b)


def _gather_max(proj, tok):
    V, Dw = proj.shape
    W = 16
    SENT = tok.shape[0] // W
    sent_per_w = SENT // NW
    NBUF = 4
    BODY = 8
    n_bodies = sent_per_w // BODY
    mesh = plsc.VectorSubcoreMesh(core_axis_name="c", subcore_axis_name="s")

    @functools.partial(
        pl.kernel,
        out_type=jax.ShapeDtypeStruct((SENT, Dw), jnp.int32),
        mesh=mesh,
        scratch_types=[
            pltpu.VMEM((sent_per_w * W,), jnp.int32),
            [pltpu.VMEM((W, Dw), jnp.int32) for _ in range(NBUF)],
            pltpu.VMEM((BODY, Dw), jnp.int32),
            [pltpu.SemaphoreType.DMA for _ in range(NBUF)],
        ],
    )
    def k(proj_hbm, tok_hbm, out_hbm, idx_v, bufs, out_v, sems):
        wid = lax.axis_index("s") * NC + lax.axis_index("c")
        base = wid * sent_per_w
        pltpu.sync_copy(tok_hbm.at[pl.ds(base * W, sent_per_w * W)], idx_v)

        def fire(local_s, b):
            idx = idx_v[pl.ds(local_s * W, W)]
            pltpu.async_copy(proj_hbm.at[idx], bufs[b], sems[b])

        def drain(b):
            pltpu.make_async_copy(
                proj_hbm.at[pl.ds(0, W)], bufs[b], sems[b]).wait()

        def reduce_sent(b, o_row):
            buf = bufs[b]

            @plsc.parallel_loop(0, Dw // 32, 1, unroll=2)
            def col_body(c):
                for half in range(2):
                    o = (2 * c + half) * 16
                    w0 = buf[0, pl.ds(o, 16)].astype(jnp.uint32)
                    m_hi = w0
                    m_lo = w0 & jnp.uint32(0xFFFF)
                    for t in range(1, W):
                        wt = buf[t, pl.ds(o, 16)].astype(jnp.uint32)
                        m_hi = jnp.maximum(m_hi, wt)
                        m_lo = jnp.maximum(m_lo, wt & jnp.uint32(0xFFFF))
                    out_v[o_row, pl.ds(o, 16)] = (
                        (m_hi & jnp.uint32(0xFFFF0000)) | m_lo
                    ).astype(jnp.int32)

        for p in range(NBUF - 1):
            fire(p, p)

        def body(q, _):
            s0 = q * BODY
            for u in range(BODY):
                b = u % NBUF

                @pl.when(s0 + u + NBUF - 1 < sent_per_w)
                def _():
                    fire(s0 + u + NBUF - 1, (u + NBUF - 1) % NBUF)

                drain(b)
                reduce_sent(b, u)

            pltpu.sync_copy(out_v, out_hbm.at[pl.ds(base + s0, BODY)])
            return 0

        lax.fori_loop(0, n_bodies, body, 0)

    return k(proj, tok)


def kernel(tokens, word_embd, fc1_w, fc1_b, fc2_w, fc2_b):
    proj = _proj_matmul(word_embd, fc1_w, fc1_b, 256)
    tok_flat = tokens.astype(jnp.int32).reshape(-1)
    pooled = _gather_max(proj, tok_flat)
    return _fc2_matmul(pooled, fc2_w, fc2_b, 512)


# speedup vs baseline: 1.0501x; 1.0029x over previous
import functools

import jax
import jax.numpy as jnp
from jax import lax
from jax.experimental import pallas as pl
from jax.experimental.pallas import tpu as pltpu
from jax.experimental.pallas import tpu_sc as plsc

NC, NS = 2, 16
NW = NC * NS


def _proj_body(x_ref, w_ref, b_ref, o_ref):
    acc = jnp.dot(x_ref[...].astype(jnp.bfloat16),
                  w_ref[...].astype(jnp.bfloat16),
                  preferred_element_type=jnp.float32) + b_ref[...]
    bf = jax.nn.relu(acc).astype(jnp.bfloat16)
    n = bf.shape[1]
    u = lax.bitcast_convert_type(bf, jnp.uint16).astype(jnp.uint32)
    word = u[:, :n // 2] | (u[:, n // 2:] << 16)
    o_ref[...] = lax.bitcast_convert_type(word, jnp.int32)


def _proj_matmul(x, w, b, bm):
    m, k = x.shape
    n = w.shape[1]
    return pl.pallas_call(
        _proj_body,
        grid=(pl.cdiv(m, bm),),
        in_specs=[
            pl.BlockSpec((bm, k), lambda i: (i, 0)),
            pl.BlockSpec((k, n), lambda i: (0, 0)),
            pl.BlockSpec((n,), lambda i: (0,)),
        ],
        out_specs=pl.BlockSpec((bm, n // 2), lambda i: (i, 0)),
        out_shape=jax.ShapeDtypeStruct((m, n // 2), jnp.int32),
    )(x, w, b)


def _fc2_body(x_ref, w_ref, b_ref, o_ref):
    xu = lax.bitcast_convert_type(x_ref[...], jnp.uint32)
    lo = (xu & jnp.uint32(0xFFFF)).astype(jnp.uint16)
    hi = (xu >> 16).astype(jnp.uint16)
    xb = jnp.concatenate(
        [lax.bitcast_convert_type(lo, jnp.bfloat16),
         lax.bitcast_convert_type(hi, jnp.bfloat16)], axis=1)
    o_ref[...] = jnp.dot(xb, w_ref[...].astype(jnp.bfloat16),
                         preferred_element_type=jnp.float32) + b_ref[...]


def _fc2_matmul(x, w, b, bm):
    m, kw = x.shape
    k, n = w.shape
    return pl.pallas_call(
        _fc2_body,
        grid=(m // bm,),
        in_specs=[
            pl.BlockSpec((bm, kw), lambda i: (i, 0)),
            pl.BlockSpec((k, n), lambda i: (0, 0)),
            pl.BlockSpec((n,), lambda i: (0,)),
        ],
        out_specs=pl.BlockSpec((bm, n), lambda i: (i, 0)),
        out_shape=jax.ShapeDtypeStruct((m, n), jnp.float32),
    )(x, w, b)


def _gather_max(proj, tok):
    V, Dw = proj.shape
    W = 16
    SENT = tok.shape[0] // W
    sent_per_w = SENT // NW
    NBUF = 4
    BODY = 8
    n_bodies = sent_per_w // BODY
    mesh = plsc.VectorSubcoreMesh(core_axis_name="c", subcore_axis_name="s")

    @functools.partial(
        pl.kernel,
        out_type=jax.ShapeDtypeStruct((SENT, Dw), jnp.int32),
        mesh=mesh,
        scratch_types=[
            pltpu.VMEM((sent_per_w * W,), jnp.int32),
            [pltpu.VMEM((W, Dw), jnp.int32) for _ in range(NBUF)],
            pltpu.VMEM((BODY, Dw), jnp.int32),
            [pltpu.SemaphoreType.DMA for _ in range(NBUF)],
            [pltpu.SemaphoreType.DMA for _ in range(2)],
        ],
    )
    def k(proj_hbm, tok_hbm, out_hbm, idx_v, bufs, out_v, sems, osems):
        wid = lax.axis_index("s") * NC + lax.axis_index("c")
        base = wid * sent_per_w
        pltpu.sync_copy(tok_hbm.at[pl.ds(base * W, sent_per_w * W)], idx_v)

        def fire(local_s, b):
            idx = idx_v[pl.ds(local_s * W, W)]
            pltpu.async_copy(proj_hbm.at[idx], bufs[b], sems[b])

        def drain(b):
            pltpu.make_async_copy(
                proj_hbm.at[pl.ds(0, W)], bufs[b], sems[b]).wait()

        def reduce_sent(b, o_row):
            buf = bufs[b]

            @plsc.parallel_loop(0, Dw // 32, 1, unroll=2)
            def col_body(c):
                for half in range(2):
                    o = (2 * c + half) * 16
                    w0 = buf[0, pl.ds(o, 16)].astype(jnp.uint32)
                    m_hi = w0
                    m_lo = w0 & jnp.uint32(0xFFFF)
                    for t in range(1, W):
                        wt = buf[t, pl.ds(o, 16)].astype(jnp.uint32)
                        m_hi = jnp.maximum(m_hi, wt)
                        m_lo = jnp.maximum(m_lo, wt & jnp.uint32(0xFFFF))
                    out_v[o_row, pl.ds(o, 16)] = (
                        (m_hi & jnp.uint32(0xFFFF0000)) | m_lo
                    ).astype(jnp.int32)

        for p in range(NBUF - 1):
            fire(p, p)

        H = BODY // 2

        def flush(half, s0):
            pltpu.async_copy(
                out_v.at[pl.ds(half * H, H)],
                out_hbm.at[pl.ds(base + s0 + half * H, H)], osems[half])

        def flush_wait(half):
            pltpu.make_async_copy(
                out_v.at[pl.ds(half * H, H)],
                out_hbm.at[pl.ds(0, H)], osems[half]).wait()

        def body(q, _):
            s0 = q * BODY
            for u in range(BODY):
                b = u % NBUF

                @pl.when(s0 + u + NBUF - 1 < sent_per_w)
                def _():
                    fire(s0 + u + NBUF - 1, (u + NBUF - 1) % NBUF)

                if u % H == 0:
                    @pl.when(q > 0)
                    def _():
                        flush_wait(u // H)

                drain(b)
                reduce_sent(b, u)
                if u % H == H - 1:
                    flush(u // H, s0)
            return 0

        lax.fori_loop(0, n_bodies, body, 0)
        flush_wait(0)
        flush_wait(1)

    return k(proj, tok)


def kernel(tokens, word_embd, fc1_w, fc1_b, fc2_w, fc2_b):
    proj = _proj_matmul(word_embd, fc1_w, fc1_b, 256)
    tok_flat = tokens.astype(jnp.int32).reshape(-1)
    pooled = _gather_max(proj, tok_flat)
    return _fc2_matmul(pooled, fc2_w, fc2_b, 512)
